# baseline (speedup 1.0000x reference)
import functools
import math

import jax
import jax.numpy as jnp
from jax import lax
from jax.experimental import pallas as pl
from jax.experimental.pallas import tpu as pltpu
from jax.experimental.pallas import tpu_sc as plsc

D_MODEL = 64
SCALE = math.sqrt(D_MODEL)

_INFO = plsc.get_sparse_core_info()
NUM_WORKERS = _INFO.num_cores * _INFO.num_subcores

NBUF = 4


def _emb_kernel(n_seq, seq_len, x_hbm, table_hbm, out_hbm, idx_v, rows_v,
                *sems):
    gsem = sems[0:NBUF]
    isem = sems[NBUF:2 * NBUF]
    osem = sems[2 * NBUF:3 * NBUF]
    wid = lax.axis_index("s") * _INFO.num_cores + lax.axis_index("c")
    per_tile = n_seq // NUM_WORKERS
    base = wid * per_tile

    def idx_start(c, b):
        pltpu.async_copy(x_hbm.at[base + c], idx_v.at[b], isem[b])

    def idx_wait(b):
        pltpu.make_async_copy(x_hbm.at[base], idx_v.at[b], isem[b]).wait()

    def gather_start(b):
        pltpu.async_copy(table_hbm.at[idx_v.at[b]], rows_v.at[b], gsem[b])

    def gather_wait(b):
        pltpu.make_async_copy(table_hbm.at[idx_v.at[b]],
                              rows_v.at[b], gsem[b]).wait()

    def ostore_start(c, b):
        pltpu.async_copy(rows_v.at[b], out_hbm.at[base + c], osem[b])

    def ostore_wait(b):
        pltpu.make_async_copy(rows_v.at[b], out_hbm.at[base], osem[b]).wait()

    def scale(b):
        def sb(j, _):
            for k in range(D_MODEL // 16):
                rows_v[b, j, pl.ds(k * 16, 16)] = (
                    rows_v[b, j, pl.ds(k * 16, 16)] * SCALE)
            return ()

        lax.fori_loop(0, seq_len, sb, (), unroll=8)

    def do_chunk(i, b, launch_gather, wait_ostore, launch_idx):
        bj = (b + 2) % NBUF
        if launch_gather:
            if wait_ostore:
                ostore_wait(bj)
            idx_wait(bj)
            gather_start(bj)
        gather_wait(b)
        if launch_idx:
            idx_start(i + 4, b)
        scale(b)
        ostore_start(i, b)

    pltpu.sync_copy(x_hbm.at[base], idx_v.at[0])
    gather_start(0)
    pltpu.sync_copy(x_hbm.at[base + 1], idx_v.at[1])
    gather_start(1)
    idx_start(2, 2)
    idx_start(3, 3)

    do_chunk(0, 0, True, False, True)
    do_chunk(1, 1, True, False, True)
    do_chunk(2, 2, True, True, True)
    do_chunk(3, 3, True, True, True)

    steps = per_tile // NBUF

    def body(s, _):
        i0 = s * NBUF
        for b in range(NBUF):
            do_chunk(i0 + b, b, True, True, True)
        return ()

    lax.fori_loop(1, steps - 1, body, ())

    n = per_tile
    do_chunk(n - 4, 0, True, True, False)
    do_chunk(n - 3, 1, True, True, False)
    do_chunk(n - 2, 2, False, False, False)
    do_chunk(n - 1, 3, False, False, False)

    for b in range(NBUF):
        ostore_wait(b)


def kernel(x, table):
    n_seq, seq_len = x.shape
    assert n_seq % (NUM_WORKERS * NBUF) == 0

    mesh = plsc.VectorSubcoreMesh(core_axis_name="c", subcore_axis_name="s")
    run = pl.kernel(
        functools.partial(_emb_kernel, n_seq, seq_len),
        out_type=jax.ShapeDtypeStruct((n_seq, seq_len, D_MODEL), jnp.float32),
        mesh=mesh,
        scratch_types=(
            [pltpu.VMEM((NBUF, seq_len), jnp.int32),
             pltpu.VMEM((NBUF, seq_len, D_MODEL), jnp.float32)]
            + [pltpu.SemaphoreType.DMA] * (3 * NBUF)
        ),
        compiler_params=pltpu.CompilerParams(use_tc_tiling_on_sc=False),
    )
    return run(x, table)
